# baseline (speedup 1.0000x reference)
import functools

import jax
import jax.numpy as jnp
from jax import lax
from jax.experimental import pallas as pl
from jax.experimental.pallas import tpu as pltpu
from jax.experimental.pallas import tpu_sc as plsc


def _proj_body(table_ref, w_ref, b_ref, s_ref):
    w8 = jnp.broadcast_to(w_ref[...], (8, w_ref.shape[1]))
    s_ref[...] = (
        lax.dot_general(
            table_ref[...], w8,
            (((1,), (1,)), ((), ())),
            preferred_element_type=jnp.float32,
        )
        + b_ref[0]
    )


def _project_table(table, W, b, block_rows=8192):
    V, D = table.shape
    grid = (V + block_rows - 1) // block_rows
    return pl.pallas_call(
        _proj_body,
        grid=(grid,),
        in_specs=[
            pl.BlockSpec((block_rows, D), lambda i: (i, 0)),
            pl.BlockSpec((1, D), lambda i: (0, 0)),
            pl.BlockSpec(memory_space=pltpu.SMEM),
        ],
        out_specs=pl.BlockSpec((block_rows, 8), lambda i: (i, 0)),
        out_shape=jax.ShapeDtypeStruct((V, 8), jnp.float32),
    )(table, W, b)


def _make_pool_kernel(L, B, n_workers, lanes):
    bw = B // n_workers
    chunks = bw // lanes
    mesh = plsc.VectorSubcoreMesh(core_axis_name="c", subcore_axis_name="s")
    nc = 2

    @functools.partial(
        pl.kernel,
        out_type=jax.ShapeDtypeStruct((B,), jnp.float32),
        mesh=mesh,
        scratch_types=[
            pltpu.VMEM((L, bw), jnp.int32),
            pltpu.VMEM((L, bw), jnp.float32),
            pltpu.VMEM((bw,), jnp.float32),
            pltpu.SemaphoreType.DMA,
        ],
    )
    def pool(s_hbm, text_hbm, out_hbm, idx_v, vals_v, res_v, sem):
        wid = lax.axis_index("s") * nc + lax.axis_index("c")
        b0 = wid * bw
        pltpu.sync_copy(text_hbm.at[:, pl.ds(b0, bw)], idx_v)
        copies = [
            pltpu.async_copy(s_hbm.at[idx_v.at[l]], vals_v.at[l], sem)
            for l in range(L)
        ]
        for c in copies:
            c.wait()
        inv_l = jnp.float32(1.0 / L)
        for j in range(chunks):
            acc = jnp.zeros((lanes,), jnp.float32)
            for l in range(L):
                acc = acc + vals_v[l, pl.ds(j * lanes, lanes)]
            res_v[pl.ds(j * lanes, lanes)] = acc * inv_l
        pltpu.sync_copy(res_v, out_hbm.at[pl.ds(b0, bw)])

    return pool


def kernel(text, table, W, b):
    L, B = text.shape
    s = _project_table(table, W, b)[:, 0]
    pool = _make_pool_kernel(L, B, n_workers=32, lanes=16)
    out = pool(s, text)
    return out.reshape(B, 1)


# speedup vs baseline: 3.3361x; 3.3361x over previous
import functools

import jax
import jax.numpy as jnp
from jax import lax
from jax.experimental import pallas as pl
from jax.experimental.pallas import tpu as pltpu
from jax.experimental.pallas import tpu_sc as plsc


def _proj_body(tableT_ref, wt_ref, b_ref, s_ref):
    s_ref[...] = (
        jnp.sum(tableT_ref[...] * wt_ref[...], axis=0, keepdims=True)
        + b_ref[0]
    )


def _project_table(tableT, Wt, b, block_cols=8192):
    D, V = tableT.shape
    grid = (V + block_cols - 1) // block_cols
    return pl.pallas_call(
        _proj_body,
        grid=(grid,),
        in_specs=[
            pl.BlockSpec((D, block_cols), lambda i: (0, i)),
            pl.BlockSpec((D, 1), lambda i: (0, 0)),
            pl.BlockSpec(memory_space=pltpu.SMEM),
        ],
        out_specs=pl.BlockSpec((1, block_cols), lambda i: (0, i)),
        out_shape=jax.ShapeDtypeStruct((1, V), jnp.float32),
    )(tableT, Wt, b)


def _make_pool_kernel(L, B, n_workers, lanes):
    bw = B // n_workers
    chunks = bw // lanes
    mesh = plsc.VectorSubcoreMesh(core_axis_name="c", subcore_axis_name="s")
    nc = 2

    @functools.partial(
        pl.kernel,
        out_type=jax.ShapeDtypeStruct((B,), jnp.float32),
        mesh=mesh,
        scratch_types=[
            pltpu.VMEM((L, bw), jnp.int32),
            pltpu.VMEM((L, bw), jnp.float32),
            pltpu.VMEM((bw,), jnp.float32),
            pltpu.SemaphoreType.DMA,
        ],
    )
    def pool(s_hbm, text_hbm, out_hbm, idx_v, vals_v, res_v, sem):
        wid = lax.axis_index("s") * nc + lax.axis_index("c")
        b0 = wid * bw
        pltpu.sync_copy(text_hbm.at[:, pl.ds(b0, bw)], idx_v)
        copies = [
            pltpu.async_copy(s_hbm.at[idx_v.at[l]], vals_v.at[l], sem)
            for l in range(L)
        ]
        for c in copies:
            c.wait()
        inv_l = jnp.float32(1.0 / L)
        for j in range(chunks):
            acc = jnp.zeros((lanes,), jnp.float32)
            for l in range(L):
                acc = acc + vals_v[l, pl.ds(j * lanes, lanes)]
            res_v[pl.ds(j * lanes, lanes)] = acc * inv_l
        pltpu.sync_copy(res_v, out_hbm.at[pl.ds(b0, bw)])

    return pool


def kernel(text, table, W, b):
    L, B = text.shape
    s = _project_table(table.T, W.T, b).reshape(-1)
    pool = _make_pool_kernel(L, B, n_workers=32, lanes=16)
    out = pool(s, text)
    return out.reshape(B, 1)


# speedup vs baseline: 3.3370x; 1.0003x over previous
import functools

import jax
import jax.numpy as jnp
from jax import lax
from jax.experimental import pallas as pl
from jax.experimental.pallas import tpu as pltpu
from jax.experimental.pallas import tpu_sc as plsc


def _proj_body(tableT_ref, wt_ref, b_ref, s_ref):
    s_ref[...] = jnp.sum(tableT_ref[...] * wt_ref[...], axis=0) + b_ref[0]


def _project_table(tableT, Wt, b, block_cols=8192):
    D, V = tableT.shape
    grid = (V + block_cols - 1) // block_cols
    return pl.pallas_call(
        _proj_body,
        grid=(grid,),
        in_specs=[
            pl.BlockSpec((D, block_cols), lambda i: (0, i)),
            pl.BlockSpec((D, 1), lambda i: (0, 0)),
            pl.BlockSpec(memory_space=pltpu.SMEM),
        ],
        out_specs=pl.BlockSpec((block_cols,), lambda i: (i,)),
        out_shape=jax.ShapeDtypeStruct((V,), jnp.float32),
    )(tableT, Wt, b)


def _make_pool_kernel(L, B, n_workers, lanes):
    bw = B // n_workers
    chunks = bw // lanes
    mesh = plsc.VectorSubcoreMesh(core_axis_name="c", subcore_axis_name="s")
    nc = 2

    @functools.partial(
        pl.kernel,
        out_type=jax.ShapeDtypeStruct((B,), jnp.float32),
        mesh=mesh,
        scratch_types=[
            pltpu.VMEM((L * bw,), jnp.int32),
            pltpu.VMEM((L * bw,), jnp.float32),
            pltpu.VMEM((bw,), jnp.float32),
            pltpu.SemaphoreType.DMA,
        ],
    )
    def pool(s_hbm, textr_hbm, out_hbm, idx_v, vals_v, res_v, sem):
        wid = lax.axis_index("s") * nc + lax.axis_index("c")
        b0 = wid * bw
        pltpu.sync_copy(textr_hbm.at[wid], idx_v)
        pltpu.async_copy(s_hbm.at[idx_v], vals_v, sem).wait()
        inv_l = jnp.float32(1.0 / L)
        for j in range(chunks):
            acc = jnp.zeros((lanes,), jnp.float32)
            for l in range(L):
                acc = acc + vals_v[pl.ds(l * bw + j * lanes, lanes)]
            res_v[pl.ds(j * lanes, lanes)] = acc * inv_l
        pltpu.sync_copy(res_v, out_hbm.at[pl.ds(b0, bw)])

    return pool


def kernel(text, table, W, b):
    L, B = text.shape
    n_workers = 32
    bw = B // n_workers
    s = _project_table(table.T, W.T, b)
    textr = text.reshape(L, n_workers, bw).transpose(1, 0, 2).reshape(n_workers, L * bw)
    pool = _make_pool_kernel(L, B, n_workers=n_workers, lanes=16)
    out = pool(s, textr)
    return out.reshape(B, 1)


# speedup vs baseline: 3.4197x; 1.0248x over previous
import functools

import jax
import jax.numpy as jnp
from jax import lax
from jax.experimental import pallas as pl
from jax.experimental.pallas import tpu as pltpu
from jax.experimental.pallas import tpu_sc as plsc


def _proj_body(tableT_ref, wt_ref, b_ref, s_ref):
    s_ref[...] = jnp.sum(tableT_ref[...] * wt_ref[...], axis=0) + b_ref[0]


def _project_table(tableT, Wt, b, block_cols=8192):
    D, V = tableT.shape
    grid = (V + block_cols - 1) // block_cols
    return pl.pallas_call(
        _proj_body,
        grid=(grid,),
        in_specs=[
            pl.BlockSpec((D, block_cols), lambda i: (0, i)),
            pl.BlockSpec((D, 1), lambda i: (0, 0)),
            pl.BlockSpec(memory_space=pltpu.SMEM),
        ],
        out_specs=pl.BlockSpec((block_cols,), lambda i: (i,)),
        out_shape=jax.ShapeDtypeStruct((V,), jnp.float32),
    )(tableT, Wt, b)


def _make_pool_kernel(L, B, V, n_workers, lanes):
    bw = B // n_workers
    chunks = bw // lanes
    mesh = plsc.VectorSubcoreMesh(core_axis_name="c", subcore_axis_name="s")
    nc = 2

    @functools.partial(
        pl.kernel,
        out_type=jax.ShapeDtypeStruct((B,), jnp.float32),
        mesh=mesh,
        scratch_types=[
            pltpu.VMEM((L, bw), jnp.int32),
            pltpu.VMEM((L, bw), jnp.float32),
            pltpu.VMEM((bw,), jnp.float32),
            pltpu.SemaphoreType.DMA,
        ],
    )
    def pool(s_hbm, text_hbm, out_hbm, idx_v, gat_v, res_v, sem):
        wid = lax.axis_index("s") * nc + lax.axis_index("c")
        b0 = wid * bw
        pltpu.sync_copy(text_hbm.at[:, pl.ds(b0, bw)], idx_v)
        cps = [
            pltpu.async_copy(s_hbm.at[idx_v.at[l]], gat_v.at[l], sem)
            for l in range(L)
        ]
        for cp in cps:
            cp.wait()
        inv_l = jnp.float32(1.0 / L)
        for j in range(chunks):
            acc = jnp.zeros((lanes,), jnp.float32)
            for l in range(L):
                acc = acc + gat_v[l, pl.ds(j * lanes, lanes)]
            res_v[pl.ds(j * lanes, lanes)] = acc * inv_l
        pltpu.sync_copy(res_v, out_hbm.at[pl.ds(b0, bw)])

    return pool


def kernel(text, table, W, b):
    L, B = text.shape
    V = table.shape[0]
    s = _project_table(table.T, W.T, b)
    pool = _make_pool_kernel(L, B, V, n_workers=32, lanes=16)
    out = pool(s, text)
    return out.reshape(B, 1)
